# Initial kernel scaffold; baseline (speedup 1.0000x reference)
import jax
import jax.numpy as jnp
from jax.experimental import pallas as pl


def kernel(reprs, track_idxs, memory, alpha):
    raise NotImplementedError("write your pallas kernel here")


# speedup vs baseline: 1.5097x; 1.5097x over previous
import functools

import jax
import jax.numpy as jnp
from jax.experimental import pallas as pl

N_TRACKS, Q, N = 256, 8, 128
B = 4096
EPS = 1e-09
TB = 64
R = TB * Q


def _update_kernel(mem_ref, reprs_ref, tids_ref, alpha_ref, out_ref):
    i = pl.program_id(0)
    mem = mem_ref[...]
    memf = mem.reshape(R, N)
    reprs = reprs_ref[...]

    sim = jax.lax.dot_general(
        memf, reprs, (((1,), (1,)), ((), ())),
        preferred_element_type=jnp.float32)

    row_t = jax.lax.broadcasted_iota(jnp.int32, (R, B), 0) // Q + i * TB
    tids = tids_ref[...].reshape(1, B)
    mask = row_t == tids
    simm = jnp.where(mask, sim, jnp.inf)

    minv = jnp.min(simm, axis=1, keepdims=True)
    bidx = jax.lax.broadcasted_iota(jnp.int32, (R, B), 1)
    idx = jnp.min(jnp.where(simm == minv, bidx, B), axis=1, keepdims=True)
    onehot = (bidx == idx).astype(jnp.float32)
    chosen = jax.lax.dot_general(
        onehot, reprs, (((1,), (0,)), ((), ())),
        preferred_element_type=jnp.float32).reshape(TB, Q, N)

    alpha = alpha_ref[...].reshape(1, Q, 1)
    new = mem * alpha + chosen * (1.0 - alpha)
    norm = jnp.sqrt(jnp.sum(new * new, axis=-1, keepdims=True))
    new = new / (norm + EPS)

    present = jnp.isfinite(minv).reshape(TB, Q, 1)
    out_ref[...] = jnp.where(present, new, mem)


@jax.jit
def kernel(reprs, track_idxs, memory, alpha):
    tids = track_idxs.astype(jnp.int32).reshape(1, B)
    grid = N_TRACKS // TB
    out = pl.pallas_call(
        _update_kernel,
        grid=(grid,),
        in_specs=[
            pl.BlockSpec((TB, Q, N), lambda i: (i, 0, 0)),
            pl.BlockSpec((B, N), lambda i: (0, 0)),
            pl.BlockSpec((1, B), lambda i: (0, 0)),
            pl.BlockSpec((Q, 1), lambda i: (0, 0)),
        ],
        out_specs=pl.BlockSpec((TB, Q, N), lambda i: (i, 0, 0)),
        out_shape=jax.ShapeDtypeStruct((N_TRACKS, Q, N), jnp.float32),
    )(memory, reprs, tids, alpha)
    return out
